# baseline (speedup 1.0000x reference)
import jax
import jax.numpy as jnp
from jax import lax
from jax.experimental import pallas as pl
from jax.experimental.pallas import tpu as pltpu
from jax.experimental.pallas import tpu_sc as plsc

BATCH = 4096
EMBED_DIM = 64
PAIR = 2 * EMBED_DIM
NUM_WORKERS = 32
CHUNK = BATCH // NUM_WORKERS


def _sc_gather_fn(u_tab_hbm, m_tab_hbm, u_idx_hbm, m_idx_hbm,
                  ue_hbm, me_hbm,
                  u_idx_v, m_idx_v, u_rows_v, m_rows_v, sem_u, sem_m):
    wid = lax.axis_index("s") * 2 + lax.axis_index("c")
    base = wid * CHUNK
    pltpu.sync_copy(u_idx_hbm.at[pl.ds(base, CHUNK)], u_idx_v)
    pltpu.sync_copy(m_idx_hbm.at[pl.ds(base, CHUNK)], m_idx_v)
    cp_u = pltpu.async_copy(u_tab_hbm.at[u_idx_v], u_rows_v, sem_u)
    cp_m = pltpu.async_copy(m_tab_hbm.at[m_idx_v], m_rows_v, sem_m)
    cp_u.wait()
    cp_m.wait()
    pltpu.sync_copy(u_rows_v, ue_hbm.at[pl.ds(base, CHUNK)])
    pltpu.sync_copy(m_rows_v, me_hbm.at[pl.ds(base, CHUNK)])


def _sc_gather(pu_idx, pm_idx, user_pairs, movie_pairs):
    mesh = plsc.VectorSubcoreMesh(core_axis_name="c", subcore_axis_name="s")
    row = jax.ShapeDtypeStruct((BATCH, PAIR), jnp.float32)
    k = pl.kernel(
        _sc_gather_fn,
        out_type=(row, row),
        mesh=mesh,
        scratch_types=[
            pltpu.VMEM((CHUNK,), jnp.int32),
            pltpu.VMEM((CHUNK,), jnp.int32),
            pltpu.VMEM((CHUNK, PAIR), jnp.float32),
            pltpu.VMEM((CHUNK, PAIR), jnp.float32),
            pltpu.SemaphoreType.DMA,
            pltpu.SemaphoreType.DMA,
        ],
    )
    return k(user_pairs, movie_pairs, pu_idx, pm_idx)


def _bn(x, g, be, eps=1e-5):
    mu = jnp.mean(x, axis=0, keepdims=True)
    var = jnp.mean((x - mu) ** 2, axis=0, keepdims=True)
    return (x - mu) * (g * lax.rsqrt(var + eps)) + be


def _mlp_fn(pue, pme, par_u, par_m, w1u, w1m, b1, g1, be1, w2, b2, g2, be2,
            w3, b3, g3, be3, wp, bp, o_ref):
    pu = pue[...]
    pm = pme[...]
    ue = jnp.where(par_u[...] > 0, pu[:, EMBED_DIM:], pu[:, :EMBED_DIM])
    me = jnp.where(par_m[...] > 0, pm[:, EMBED_DIM:], pm[:, :EMBED_DIM])
    x = jnp.dot(ue, w1u[...], preferred_element_type=jnp.float32)
    x = x + jnp.dot(me, w1m[...], preferred_element_type=jnp.float32)
    x = jnp.maximum(x + b1[...], 0.0)
    x = _bn(x, g1[...], be1[...])
    x = jnp.dot(x, w2[...], preferred_element_type=jnp.float32)
    x = jnp.maximum(x + b2[...], 0.0)
    x = _bn(x, g2[...], be2[...])
    x = jnp.dot(x, w3[...], preferred_element_type=jnp.float32)
    x = jnp.maximum(x + b3[...], 0.0)
    x = _bn(x, g3[...], be3[...])
    p = jnp.sum(x * wp[...], axis=1, keepdims=True) + bp[...]
    o_ref[...] = jax.nn.sigmoid(p) * 5.0


def _tc_mlp(pue, pme, par_u, par_m,
            W1, b1, g1, be1, W2, b2, g2, be2, W3, b3, g3, be3, Wp, bp):
    return pl.pallas_call(
        _mlp_fn,
        out_shape=jax.ShapeDtypeStruct((BATCH, 1), jnp.float32),
    )(pue, pme, par_u, par_m,
      W1[:EMBED_DIM], W1[EMBED_DIM:],
      b1.reshape(1, -1), g1.reshape(1, -1), be1.reshape(1, -1),
      W2, b2.reshape(1, -1), g2.reshape(1, -1), be2.reshape(1, -1),
      W3, b3.reshape(1, -1), g3.reshape(1, -1), be3.reshape(1, -1),
      Wp.reshape(1, -1), bp.reshape(1, 1))


def kernel(users, movies, user_table, movie_table,
           W1, b1, g1, be1, W2, b2, g2, be2, W3, b3, g3, be3, Wp, bp):
    u = users.astype(jnp.int32)
    m = movies.astype(jnp.int32)
    pue, pme = _sc_gather(u >> 1, m >> 1,
                          user_table.reshape(-1, PAIR),
                          movie_table.reshape(-1, PAIR))
    par_u = (u & 1).reshape(BATCH, 1)
    par_m = (m & 1).reshape(BATCH, 1)
    return _tc_mlp(pue, pme, par_u, par_m, W1, b1, g1, be1,
                   W2, b2, g2, be2, W3, b3, g3, be3, Wp, bp)


# speedup vs baseline: 1.1626x; 1.1626x over previous
import jax
import jax.numpy as jnp
from jax import lax
from jax.experimental import pallas as pl
from jax.experimental.pallas import tpu as pltpu
from jax.experimental.pallas import tpu_sc as plsc

BATCH = 4096
EMBED_DIM = 64
PAIR = 2 * EMBED_DIM
HALF = 50176
NUM_WORKERS = 32
CHUNK = BATCH // NUM_WORKERS


def _tpose_fn(x1_ref, x2_ref, o_ref):
    o_ref[...] = jnp.concatenate([x1_ref[...].T, x2_ref[...].T], axis=1)


def _tc_pack_transpose(tab_t):
    return pl.pallas_call(
        _tpose_fn,
        grid=(49,),
        in_specs=[
            pl.BlockSpec((64, 1024), lambda i: (0, i)),
            pl.BlockSpec((64, 1024), lambda i: (0, i + 49)),
        ],
        out_specs=pl.BlockSpec((1024, PAIR), lambda i: (i, 0)),
        out_shape=jax.ShapeDtypeStruct((HALF, PAIR), jnp.float32),
    )(tab_t, tab_t)


def _sc_gather_fn(tab_hbm, idx_hbm, out_hbm, idx_v, rows_v, sem):
    wid = lax.axis_index("s") * 2 + lax.axis_index("c")
    base = wid * CHUNK
    pltpu.sync_copy(idx_hbm.at[pl.ds(base, CHUNK)], idx_v)
    pltpu.async_copy(tab_hbm.at[idx_v], rows_v, sem).wait()
    pltpu.sync_copy(rows_v, out_hbm.at[pl.ds(base, CHUNK)])


def _sc_gather(pidx, packed):
    mesh = plsc.VectorSubcoreMesh(core_axis_name="c", subcore_axis_name="s")
    k = pl.kernel(
        _sc_gather_fn,
        out_type=jax.ShapeDtypeStruct((BATCH, PAIR), jnp.float32),
        mesh=mesh,
        scratch_types=[
            pltpu.VMEM((CHUNK,), jnp.int32),
            pltpu.VMEM((CHUNK, PAIR), jnp.float32),
            pltpu.SemaphoreType.DMA,
        ],
    )
    return k(packed, pidx)


def _bn(x, g, be, eps=1e-5):
    mu = jnp.mean(x, axis=0, keepdims=True)
    var = jnp.mean((x - mu) ** 2, axis=0, keepdims=True)
    return (x - mu) * (g * lax.rsqrt(var + eps)) + be


def _mlp_fn(pue, pme, par_u, par_m, w1u, w1m, b1, g1, be1, w2, b2, g2, be2,
            w3, b3, g3, be3, wp, bp, o_ref):
    pu = pue[...]
    pm = pme[...]
    ue = jnp.where(par_u[...] > 0, pu[:, EMBED_DIM:], pu[:, :EMBED_DIM])
    me = jnp.where(par_m[...] > 0, pm[:, EMBED_DIM:], pm[:, :EMBED_DIM])
    x = jnp.dot(ue, w1u[...], preferred_element_type=jnp.float32)
    x = x + jnp.dot(me, w1m[...], preferred_element_type=jnp.float32)
    x = jnp.maximum(x + b1[...], 0.0)
    x = _bn(x, g1[...], be1[...])
    x = jnp.dot(x, w2[...], preferred_element_type=jnp.float32)
    x = jnp.maximum(x + b2[...], 0.0)
    x = _bn(x, g2[...], be2[...])
    x = jnp.dot(x, w3[...], preferred_element_type=jnp.float32)
    x = jnp.maximum(x + b3[...], 0.0)
    x = _bn(x, g3[...], be3[...])
    p = jnp.sum(x * wp[...], axis=1, keepdims=True) + bp[...]
    o_ref[...] = jax.nn.sigmoid(p) * 5.0


def _tc_mlp(pue, pme, par_u, par_m,
            W1, b1, g1, be1, W2, b2, g2, be2, W3, b3, g3, be3, Wp, bp):
    return pl.pallas_call(
        _mlp_fn,
        out_shape=jax.ShapeDtypeStruct((BATCH, 1), jnp.float32),
    )(pue, pme, par_u, par_m,
      W1[:EMBED_DIM], W1[EMBED_DIM:],
      b1.reshape(1, -1), g1.reshape(1, -1), be1.reshape(1, -1),
      W2, b2.reshape(1, -1), g2.reshape(1, -1), be2.reshape(1, -1),
      W3, b3.reshape(1, -1), g3.reshape(1, -1), be3.reshape(1, -1),
      Wp.reshape(1, -1), bp.reshape(1, 1))


def kernel(users, movies, user_table, movie_table,
           W1, b1, g1, be1, W2, b2, g2, be2, W3, b3, g3, be3, Wp, bp):
    u = users.astype(jnp.int32)
    m = movies.astype(jnp.int32)
    packed_u = _tc_pack_transpose(user_table.T)
    packed_m = _tc_pack_transpose(movie_table.T)
    pue = _sc_gather(jnp.where(u < HALF, u, u - HALF), packed_u)
    pme = _sc_gather(jnp.where(m < HALF, m, m - HALF), packed_m)
    par_u = (u >= HALF).astype(jnp.int32).reshape(BATCH, 1)
    par_m = (m >= HALF).astype(jnp.int32).reshape(BATCH, 1)
    return _tc_mlp(pue, pme, par_u, par_m, W1, b1, g1, be1,
                   W2, b2, g2, be2, W3, b3, g3, be3, Wp, bp)
